# baseline (device time: 77473 ns/iter reference)
import jax
import jax.numpy as jnp
from jax import lax
from jax.experimental import pallas as pl
from jax.experimental.pallas import tpu as pltpu

N_DEV = 8
SCALE = 0.08838834764831843


def _virt(p):
    return p ^ ((p >> 1) & 1)


def kernel(x, Wq, Wo, K_ext, V_ext):
    B, Sq, D = x.shape
    Dh = K_ext.shape[3]
    Hl = K_ext.shape[2]
    Skv = K_ext.shape[1]

    x_bf = x.astype(jnp.bfloat16)
    wq_bf = (Wq * SCALE).astype(jnp.bfloat16)
    wo_bf = Wo.astype(jnp.bfloat16)
    k_flat = K_ext.reshape(B, Skv, Hl * Dh).astype(jnp.bfloat16)
    v_flat = V_ext.reshape(B, Skv, Hl * Dh).astype(jnp.bfloat16)

    R = B * Sq
    GROUPS = ((0, 384), (384, 384), (768, 256))
    ORDERS = ((4, 2, 1), (2, 1, 4), (1, 4, 2))
    NG = len(GROUPS)

    rs_off = []
    off = 0
    for g, (_, nr) in enumerate(GROUPS):
        per = []
        for si in range(3):
            per.append(off)
            off += nr >> (si + 1)
        rs_off.append(per)
    RS_ROWS = off

    def body(x_ref, wq_ref, wo_ref, k_ref, v_ref, out_ref,
             abuf, acc_ref, rs_ref, send_sems, recv_sems):
        b = pl.program_id(0)
        ab = abuf.at[b % 2]

        qb = jnp.dot(x_ref[0], wq_ref[...],
                     preferred_element_type=jnp.float32
                     ).astype(jnp.bfloat16)
        for h in range(Hl):
            cols = slice(h * Dh, (h + 1) * Dh)
            q = qb[:, cols]
            k = k_ref[0][:, cols]
            s = lax.dot_general(q, k, (((1,), (1,)), ((), ())),
                                preferred_element_type=jnp.float32)
            pexp = jnp.exp(s.astype(jnp.bfloat16))
            l = jnp.sum(pexp.astype(jnp.float32), axis=1, keepdims=True)
            o = jnp.dot(pexp, v_ref[0][:, cols],
                        preferred_element_type=jnp.float32) * (1.0 / l)
            ab[:, cols] = o.astype(jnp.bfloat16)

        acc_ref[pl.ds(b * Sq, Sq), :] = jnp.dot(
            ab[...], wo_ref[...],
            preferred_element_type=jnp.float32,
        ).astype(jnp.bfloat16)

        @pl.when(b == B - 1)
        def _():
            p = lax.axis_index("i")
            r = _virt(p)

            barrier = pltpu.get_barrier_semaphore()
            for msk in (1, 2, 4):
                pl.semaphore_signal(
                    barrier, inc=1,
                    device_id=(_virt(r ^ msk),),
                    device_id_type=pl.DeviceIdType.MESH,
                )
            pl.semaphore_wait(barrier, 3)

            los = [jnp.int32(r0) for r0, _ in GROUPS]
            for si in range(3):
                pending = []
                for g, (_, nr) in enumerate(GROUPS):
                    msk = ORDERS[g][si]
                    half = nr >> (si + 1)
                    partner = _virt(r ^ msk)
                    keep_hi = (r & msk) != 0
                    send_row = los[g] + jnp.where(keep_hi, 0, half)
                    keep_row = los[g] + jnp.where(keep_hi, half, 0)
                    rdma = pltpu.make_async_remote_copy(
                        src_ref=acc_ref.at[pl.ds(send_row, half), :],
                        dst_ref=rs_ref.at[pl.ds(rs_off[g][si], half), :],
                        send_sem=send_sems.at[g, si],
                        recv_sem=recv_sems.at[g, si],
                        device_id=(partner,),
                        device_id_type=pl.DeviceIdType.MESH,
                    )
                    rdma.start()
                    pending.append((rdma, keep_row, half, rs_off[g][si]))
                    los[g] = keep_row
                for rdma, keep_row, half, o in pending:
                    rdma.wait()
                    acc_ref[pl.ds(keep_row, half), :] = (
                        acc_ref[pl.ds(keep_row, half), :]
                        + rs_ref[pl.ds(o, half), :]
                    )

            curs = [nr >> 3 for _, nr in GROUPS]
            for si in range(3):
                pending = []
                for g in range(NG):
                    msk = ORDERS[g][2 - si]
                    cur = curs[g]
                    partner = _virt(r ^ msk)
                    rdma = pltpu.make_async_remote_copy(
                        src_ref=acc_ref.at[pl.ds(los[g], cur), :],
                        dst_ref=acc_ref.at[pl.ds(los[g], cur), :],
                        send_sem=send_sems.at[g, 3 + si],
                        recv_sem=recv_sems.at[g, 3 + si],
                        device_id=(partner,),
                        device_id_type=pl.DeviceIdType.MESH,
                    )
                    rdma.start()
                    pending.append(rdma)
                    los[g] = los[g] - jnp.where((r & msk) != 0, cur, 0)
                    curs[g] = 2 * cur
                for rdma in pending:
                    rdma.wait()

            out_ref[...] = acc_ref[...].astype(jnp.float32)

    out = pl.pallas_call(
        body,
        grid=(B,),
        in_specs=[
            pl.BlockSpec((1, Sq, D), lambda b: (b, 0, 0)),
            pl.BlockSpec((D, Hl * Dh), lambda b: (0, 0)),
            pl.BlockSpec((Hl * Dh, D), lambda b: (0, 0)),
            pl.BlockSpec((1, Skv, Hl * Dh), lambda b: (b, 0, 0)),
            pl.BlockSpec((1, Skv, Hl * Dh), lambda b: (b, 0, 0)),
        ],
        out_specs=pl.BlockSpec((R, D), lambda b: (0, 0)),
        out_shape=jax.ShapeDtypeStruct((R, D), jnp.float32),
        scratch_shapes=[
            pltpu.VMEM((2, Sq, Hl * Dh), jnp.bfloat16),
            pltpu.VMEM((R, D), jnp.bfloat16),
            pltpu.VMEM((RS_ROWS, D), jnp.bfloat16),
            pltpu.SemaphoreType.DMA((NG, 6)),
            pltpu.SemaphoreType.DMA((NG, 6)),
        ],
        compiler_params=pltpu.CompilerParams(collective_id=0),
    )(x_bf, wq_bf, wo_bf, k_flat, v_flat)

    return out.reshape(B, Sq, D)


# device time: 76457 ns/iter; 1.0133x vs baseline; 1.0133x over previous
import jax
import jax.numpy as jnp
from jax import lax
from jax.experimental import pallas as pl
from jax.experimental.pallas import tpu as pltpu

N_DEV = 8
SCALE = 0.08838834764831843


def _virt(p):
    return p ^ ((p >> 1) & 1)


def kernel(x, Wq, Wo, K_ext, V_ext):
    B, Sq, D = x.shape
    Dh = K_ext.shape[3]
    Hl = K_ext.shape[2]
    Skv = K_ext.shape[1]

    x_bf = x.astype(jnp.bfloat16)
    wq_bf = (Wq * SCALE).astype(jnp.bfloat16)
    wo_bf = Wo.astype(jnp.bfloat16)
    k_flat = K_ext.reshape(B, Skv, Hl * Dh).astype(jnp.bfloat16)
    v_flat = V_ext.reshape(B, Skv, Hl * Dh).astype(jnp.bfloat16)

    R = B * Sq
    GROUPS = ((0, 384), (384, 384), (768, 256))
    ORDERS = ((4, 2, 1), (2, 1, 4), (1, 4, 2))
    NG = len(GROUPS)

    rs_off = []
    off = 0
    for g, (_, nr) in enumerate(GROUPS):
        per = []
        for si in range(3):
            per.append(off)
            off += nr >> (si + 1)
        rs_off.append(per)
    RS_ROWS = off

    def body(x_ref, wq_ref, wo_ref, k_ref, v_ref, out_ref,
             abuf, rs_ref, send_sems, recv_sems):
        acc_ref = out_ref
        b = pl.program_id(0)
        ab = abuf.at[b % 2]

        qb = jnp.dot(x_ref[0], wq_ref[...],
                     preferred_element_type=jnp.float32
                     ).astype(jnp.bfloat16)
        for h in range(Hl):
            cols = slice(h * Dh, (h + 1) * Dh)
            q = qb[:, cols]
            k = k_ref[0][:, cols]
            s = lax.dot_general(q, k, (((1,), (1,)), ((), ())),
                                preferred_element_type=jnp.float32)
            pexp = jnp.exp(s.astype(jnp.bfloat16))
            l = jnp.sum(pexp.astype(jnp.float32), axis=1, keepdims=True)
            o = jnp.dot(pexp, v_ref[0][:, cols],
                        preferred_element_type=jnp.float32) * (1.0 / l)
            ab[:, cols] = o.astype(jnp.bfloat16)

        acc_ref[pl.ds(b * Sq, Sq), :] = jnp.dot(
            ab[...], wo_ref[...],
            preferred_element_type=jnp.float32,
        ).astype(jnp.bfloat16)

        @pl.when(b == B - 1)
        def _():
            p = lax.axis_index("i")
            r = _virt(p)

            barrier = pltpu.get_barrier_semaphore()
            for msk in (1, 2, 4):
                pl.semaphore_signal(
                    barrier, inc=1,
                    device_id=(_virt(r ^ msk),),
                    device_id_type=pl.DeviceIdType.MESH,
                )
            pl.semaphore_wait(barrier, 3)

            los = [jnp.int32(r0) for r0, _ in GROUPS]
            curs = [nr >> 3 for _, nr in GROUPS]

            def mk_rs(g, si):
                nr = GROUPS[g][1]
                msk = ORDERS[g][si]
                half = nr >> (si + 1)
                partner = _virt(r ^ msk)
                keep_hi = (r & msk) != 0
                send_row = los[g] + jnp.where(keep_hi, 0, half)
                keep_row = los[g] + jnp.where(keep_hi, half, 0)
                rdma = pltpu.make_async_remote_copy(
                    src_ref=acc_ref.at[pl.ds(send_row, half), :],
                    dst_ref=rs_ref.at[pl.ds(rs_off[g][si], half), :],
                    send_sem=send_sems.at[g, si],
                    recv_sem=recv_sems.at[g, si],
                    device_id=(partner,),
                    device_id_type=pl.DeviceIdType.MESH,
                )
                los[g] = keep_row
                return rdma, keep_row, half, rs_off[g][si]

            def mk_ag(g, si):
                msk = ORDERS[g][2 - si]
                cur = curs[g]
                partner = _virt(r ^ msk)
                rdma = pltpu.make_async_remote_copy(
                    src_ref=acc_ref.at[pl.ds(los[g], cur), :],
                    dst_ref=acc_ref.at[pl.ds(los[g], cur), :],
                    send_sem=send_sems.at[g, 3 + si],
                    recv_sem=recv_sems.at[g, 3 + si],
                    device_id=(partner,),
                    device_id_type=pl.DeviceIdType.MESH,
                )
                los[g] = los[g] - jnp.where((r & msk) != 0, cur, 0)
                curs[g] = 2 * cur
                return rdma

            inflight = [mk_rs(g, 0) for g in range(NG)]
            for g in range(NG):
                inflight[g][0].start()
            ag_inflight = [None] * NG
            for si in range(3):
                for g in range(NG):
                    rdma, keep_row, half, o = inflight[g]
                    rdma.wait()
                    acc_ref[pl.ds(keep_row, half), :] = (
                        acc_ref[pl.ds(keep_row, half), :]
                        + rs_ref[pl.ds(o, half), :]
                    )
                    if si < 2:
                        inflight[g] = mk_rs(g, si + 1)
                        inflight[g][0].start()
                    else:
                        ag_inflight[g] = mk_ag(g, 0)
                        ag_inflight[g].start()
            for si in range(3):
                for g in range(NG):
                    ag_inflight[g].wait()
                    if si < 2:
                        ag_inflight[g] = mk_ag(g, si + 1)
                        ag_inflight[g].start()

    out = pl.pallas_call(
        body,
        grid=(B,),
        in_specs=[
            pl.BlockSpec((1, Sq, D), lambda b: (b, 0, 0)),
            pl.BlockSpec((D, Hl * Dh), lambda b: (0, 0)),
            pl.BlockSpec((Hl * Dh, D), lambda b: (0, 0)),
            pl.BlockSpec((1, Skv, Hl * Dh), lambda b: (b, 0, 0)),
            pl.BlockSpec((1, Skv, Hl * Dh), lambda b: (b, 0, 0)),
        ],
        out_specs=pl.BlockSpec((R, D), lambda b: (0, 0)),
        out_shape=jax.ShapeDtypeStruct((R, D), jnp.bfloat16),
        scratch_shapes=[
            pltpu.VMEM((2, Sq, Hl * Dh), jnp.bfloat16),
            pltpu.VMEM((RS_ROWS, D), jnp.bfloat16),
            pltpu.SemaphoreType.DMA((NG, 6)),
            pltpu.SemaphoreType.DMA((NG, 6)),
        ],
        compiler_params=pltpu.CompilerParams(collective_id=0),
    )(x_bf, wq_bf, wo_bf, k_flat, v_flat)

    return out.reshape(B, Sq, D)


# device time: 76373 ns/iter; 1.0144x vs baseline; 1.0011x over previous
import jax
import jax.numpy as jnp
from jax import lax
from jax.experimental import pallas as pl
from jax.experimental.pallas import tpu as pltpu

N_DEV = 8
SCALE = 0.08838834764831843


def _virt(p):
    return p ^ ((p >> 1) & 1)


def kernel(x, Wq, Wo, K_ext, V_ext):
    B, Sq, D = x.shape
    Dh = K_ext.shape[3]
    Hl = K_ext.shape[2]
    Skv = K_ext.shape[1]

    x_bf = x.astype(jnp.bfloat16)
    wq_bf = (Wq * SCALE).astype(jnp.bfloat16)
    wo_bf = Wo.astype(jnp.bfloat16)
    k_flat = K_ext.astype(jnp.bfloat16).reshape(B, Skv, Hl * Dh)
    v_flat = V_ext.astype(jnp.bfloat16).reshape(B, Skv, Hl * Dh)

    R = B * Sq
    GROUPS = ((0, 384), (384, 384), (768, 256))
    ORDERS = ((4, 2, 1), (2, 1, 4), (1, 4, 2))
    NG = len(GROUPS)

    rs_off = []
    off = 0
    for g, (_, nr) in enumerate(GROUPS):
        per = []
        for si in range(3):
            per.append(off)
            off += nr >> (si + 1)
        rs_off.append(per)
    RS_ROWS = off

    def body(x_ref, wq_ref, wo_ref, k_ref, v_ref, out_ref,
             abuf, rs_ref, send_sems, recv_sems):
        acc_ref = out_ref
        b = pl.program_id(0)
        ab = abuf.at[b % 2]

        qb = jnp.dot(x_ref[0], wq_ref[...],
                     preferred_element_type=jnp.float32
                     ).astype(jnp.bfloat16)
        for h in range(Hl):
            cols = slice(h * Dh, (h + 1) * Dh)
            q = qb[:, cols]
            k = k_ref[0][:, cols]
            s = lax.dot_general(q, k, (((1,), (1,)), ((), ())),
                                preferred_element_type=jnp.float32)
            pexp = jnp.exp(s.astype(jnp.bfloat16))
            l = jnp.sum(pexp.astype(jnp.float32), axis=1, keepdims=True)
            o = jnp.dot(pexp, v_ref[0][:, cols],
                        preferred_element_type=jnp.float32) * (1.0 / l)
            ab[:, cols] = o.astype(jnp.bfloat16)

        acc_ref[pl.ds(b * Sq, Sq), :] = jnp.dot(
            ab[...], wo_ref[...],
            preferred_element_type=jnp.float32,
        ).astype(jnp.bfloat16)

        @pl.when(b == B - 1)
        def _():
            p = lax.axis_index("i")
            r = _virt(p)

            barrier = pltpu.get_barrier_semaphore()
            for msk in (1, 2, 4):
                pl.semaphore_signal(
                    barrier, inc=1,
                    device_id=(_virt(r ^ msk),),
                    device_id_type=pl.DeviceIdType.MESH,
                )
            pl.semaphore_wait(barrier, 3)

            los = [jnp.int32(r0) for r0, _ in GROUPS]
            curs = [nr >> 3 for _, nr in GROUPS]

            def mk_rs(g, si):
                nr = GROUPS[g][1]
                msk = ORDERS[g][si]
                half = nr >> (si + 1)
                partner = _virt(r ^ msk)
                keep_hi = (r & msk) != 0
                send_row = los[g] + jnp.where(keep_hi, 0, half)
                keep_row = los[g] + jnp.where(keep_hi, half, 0)
                rdma = pltpu.make_async_remote_copy(
                    src_ref=acc_ref.at[pl.ds(send_row, half), :],
                    dst_ref=rs_ref.at[pl.ds(rs_off[g][si], half), :],
                    send_sem=send_sems.at[g, si],
                    recv_sem=recv_sems.at[g, si],
                    device_id=(partner,),
                    device_id_type=pl.DeviceIdType.MESH,
                )
                los[g] = keep_row
                return rdma, keep_row, half, rs_off[g][si]

            def mk_ag(g, si):
                msk = ORDERS[g][2 - si]
                cur = curs[g]
                partner = _virt(r ^ msk)
                rdma = pltpu.make_async_remote_copy(
                    src_ref=acc_ref.at[pl.ds(los[g], cur), :],
                    dst_ref=acc_ref.at[pl.ds(los[g], cur), :],
                    send_sem=send_sems.at[g, 3 + si],
                    recv_sem=recv_sems.at[g, 3 + si],
                    device_id=(partner,),
                    device_id_type=pl.DeviceIdType.MESH,
                )
                los[g] = los[g] - jnp.where((r & msk) != 0, cur, 0)
                curs[g] = 2 * cur
                return rdma

            inflight = [mk_rs(g, 0) for g in range(NG)]
            for g in range(NG):
                inflight[g][0].start()
            ag_inflight = [None] * NG
            for si in range(3):
                for g in range(NG):
                    rdma, keep_row, half, o = inflight[g]
                    rdma.wait()
                    acc_ref[pl.ds(keep_row, half), :] = (
                        acc_ref[pl.ds(keep_row, half), :]
                        + rs_ref[pl.ds(o, half), :]
                    )
                    if si < 2:
                        inflight[g] = mk_rs(g, si + 1)
                        inflight[g][0].start()
                    else:
                        ag_inflight[g] = mk_ag(g, 0)
                        ag_inflight[g].start()
            for si in range(3):
                for g in range(NG):
                    ag_inflight[g].wait()
                    if si < 2:
                        ag_inflight[g] = mk_ag(g, si + 1)
                        ag_inflight[g].start()

    out = pl.pallas_call(
        body,
        grid=(B,),
        in_specs=[
            pl.BlockSpec((1, Sq, D), lambda b: (b, 0, 0)),
            pl.BlockSpec((D, Hl * Dh), lambda b: (0, 0)),
            pl.BlockSpec((Hl * Dh, D), lambda b: (0, 0)),
            pl.BlockSpec((1, Skv, Hl * Dh), lambda b: (b, 0, 0)),
            pl.BlockSpec((1, Skv, Hl * Dh), lambda b: (b, 0, 0)),
        ],
        out_specs=pl.BlockSpec((R, D), lambda b: (0, 0)),
        out_shape=jax.ShapeDtypeStruct((R, D), jnp.bfloat16),
        scratch_shapes=[
            pltpu.VMEM((2, Sq, Hl * Dh), jnp.bfloat16),
            pltpu.VMEM((RS_ROWS, D), jnp.bfloat16),
            pltpu.SemaphoreType.DMA((NG, 6)),
            pltpu.SemaphoreType.DMA((NG, 6)),
        ],
        compiler_params=pltpu.CompilerParams(collective_id=0),
    )(x_bf, wq_bf, wo_bf, k_flat, v_flat)

    return out.reshape(B, Sq, D)
